# Initial kernel scaffold; baseline (speedup 1.0000x reference)
import jax
import jax.numpy as jnp
from jax.experimental import pallas as pl


def kernel(input_ids, last_token_pos, tok_emb, ax1, ax2, Wqk, Wv, Wo, ln1g, ln1b, ln2g, ln2b, W1, b1, W2, b2, lnfg, lnfb, lm_w, lm_b, rot):
    raise NotImplementedError("write your pallas kernel here")


# speedup vs baseline: 5.9119x; 5.9119x over previous
import functools

import jax
import jax.numpy as jnp
from jax import lax
from jax.experimental import pallas as pl
from jax.experimental.pallas import tpu as pltpu
from jax.experimental.pallas import tpu_sc as plsc

B, S, D, H, L, FF, V = 1, 2048, 768, 12, 4, 2048, 128
CHUNK = 64
NB = 32
DH = D // H
NW = 32
SPW = S // NW

f32 = jnp.float32
i32 = jnp.int32


def _embed_body(ids_ref, tok_ref, ax1_ref, ax2_ref, x_ref, cnt_ref):
    ids = ids_ref[...]
    lane_v = lax.broadcasted_iota(i32, (S, V), 1)
    oh = (ids == lane_v).astype(f32)
    x = jnp.dot(oh, tok_ref[...], preferred_element_type=f32)
    su = lax.broadcasted_iota(i32, (S, 32), 0)
    la = lax.broadcasted_iota(i32, (S, 32), 1)
    oh1 = ((su // 64) == la).astype(f32)
    p1 = jnp.dot(oh1, ax1_ref[...], preferred_element_type=f32)
    su2 = lax.broadcasted_iota(i32, (S, 64), 0)
    la2 = lax.broadcasted_iota(i32, (S, 64), 1)
    oh2 = ((su2 % 64) == la2).astype(f32)
    p2 = jnp.dot(oh2, ax2_ref[...], preferred_element_type=f32)
    x_ref[...] = x + jnp.concatenate([p1, p2], axis=1)
    cnt_ref[...] = jnp.sum((ids != 0).astype(i32)).reshape(1, 1)


def _embed(ids_col, tok_emb, ax1, ax2, interpret=False):
    return pl.pallas_call(
        _embed_body,
        out_shape=[jax.ShapeDtypeStruct((S, D), f32),
                   jax.ShapeDtypeStruct((1, 1), i32)],
        interpret=interpret,
    )(ids_col, tok_emb, ax1, ax2)


def _prep_body(x_ref, ids_ref, g_ref, b_ref, wqk_ref, wv_ref, rot_ref,
               qv_ref, inv_ref):
    h = pl.program_id(0)
    x = x_ref[...]
    mu = jnp.mean(x, axis=1, keepdims=True)
    var = jnp.mean((x - mu) ** 2, axis=1, keepdims=True)
    xn = (x - mu) / jnp.sqrt(var + 1e-6) * g_ref[...] + b_ref[...]
    q = jnp.dot(xn, wqk_ref[0], preferred_element_type=f32)
    v = jnp.dot(xn, wv_ref[0], preferred_element_type=f32)
    qv_ref[0] = jnp.concatenate([q, v], axis=1)
    proj = jnp.dot(q, rot_ref[0], preferred_element_type=f32)
    c = jnp.concatenate([proj, -proj], axis=1)
    m = jnp.max(c, axis=1, keepdims=True)
    la = lax.broadcasted_iota(i32, (S, NB), 1)
    bidx = jnp.min(jnp.where(c == m, la, NB * 2), axis=1, keepdims=True)
    bucket = jnp.where(ids_ref[...] != 0, bidx, NB)
    la64 = lax.broadcasted_iota(i32, (S, 64), 1)
    oh = (bucket == la64).astype(f32)
    counts = jnp.sum(oh, axis=0, keepdims=True)
    br = lax.broadcasted_iota(i32, (64, 64), 0)
    bc = lax.broadcasted_iota(i32, (64, 64), 1)
    start = jnp.dot(counts, (br < bc).astype(f32),
                    preferred_element_type=f32)
    tr = lax.broadcasted_iota(i32, (128, 128), 0)
    tc = lax.broadcasted_iota(i32, (128, 128), 1)
    tri = (tr >= tc).astype(f32)
    carry = jnp.zeros((1, 64), f32)
    for i in range(S // 128):
        blk = oh[i * 128:(i + 1) * 128]
        cum = jnp.dot(tri, blk, preferred_element_type=f32) + carry
        invb = jnp.sum((cum + start) * blk, axis=1, keepdims=True) - 1.0
        inv_ref[0, i * 128:(i + 1) * 128, :] = invb.astype(i32) + h * S
        carry = carry + jnp.sum(blk, axis=0, keepdims=True)


def _prep(x, ids_col, g, b, wqk, wv, rot_l, interpret=False):
    return pl.pallas_call(
        _prep_body,
        grid=(H,),
        in_specs=[
            pl.BlockSpec((S, D), lambda h: (0, 0)),
            pl.BlockSpec((S, 1), lambda h: (0, 0)),
            pl.BlockSpec((1, D), lambda h: (0, 0)),
            pl.BlockSpec((1, D), lambda h: (0, 0)),
            pl.BlockSpec((1, D, DH), lambda h: (h, 0, 0)),
            pl.BlockSpec((1, D, DH), lambda h: (h, 0, 0)),
            pl.BlockSpec((1, DH, NB // 2), lambda h: (h, 0, 0)),
        ],
        out_specs=[
            pl.BlockSpec((1, S, 2 * DH), lambda h: (h, 0, 0)),
            pl.BlockSpec((1, S, 1), lambda h: (h, 0, 0)),
        ],
        out_shape=[jax.ShapeDtypeStruct((H, S, 2 * DH), f32),
                   jax.ShapeDtypeStruct((H, S, 1), i32)],
        interpret=interpret,
    )(x, ids_col, g, b, wqk, wv, rot_l)


@functools.lru_cache(maxsize=1)
def _sc_kernels():
    mesh = plsc.VectorSubcoreMesh(core_axis_name="c", subcore_axis_name="s")

    @functools.partial(
        pl.kernel,
        out_type=[jax.ShapeDtypeStruct((H * S, 2 * DH), f32),
                  jax.ShapeDtypeStruct((H, S), i32)],
        mesh=mesh,
        scratch_types=[pltpu.VMEM((SPW,), i32),
                       pltpu.VMEM((SPW, 2 * DH), f32),
                       pltpu.VMEM((S,), i32),
                       pltpu.VMEM((S,), i32),
                       pltpu.SemaphoreType.DMA],
        compiler_params=pltpu.CompilerParams(needs_layout_passes=False),
    )
    def _sc_scatter(qv_hbm, invf_hbm, qvs_hbm, perm_hbm,
                    idxbuf, rows, invfull, permbuf, sem):
        wid = lax.axis_index("s") * 2 + lax.axis_index("c")
        base = wid * SPW
        for h in range(H):
            pltpu.sync_copy(invf_hbm.at[h, pl.ds(base, SPW)], idxbuf)
            pltpu.sync_copy(qv_hbm.at[pl.ds(h * S + base, SPW), :], rows)
            pltpu.async_copy(rows, qvs_hbm.at[idxbuf], sem).wait()

        @pl.when(wid < H)
        def _():
            pltpu.sync_copy(invf_hbm.at[wid], invfull)
            for i in range(S // 16):
                idx16 = invfull[pl.ds(i * 16, 16)] - wid * S
                vals = lax.iota(i32, 16) + i * 16
                plsc.store_scatter(permbuf, [idx16], vals)
            pltpu.sync_copy(permbuf, perm_hbm.at[wid])

    @functools.partial(
        pl.kernel,
        out_type=jax.ShapeDtypeStruct((H * S, 2 * DH), f32),
        mesh=mesh,
        scratch_types=[pltpu.VMEM((SPW,), i32),
                       pltpu.VMEM((SPW, 2 * DH), f32),
                       pltpu.SemaphoreType.DMA],
        compiler_params=pltpu.CompilerParams(needs_layout_passes=False),
    )
    def _sc_gather(outs_hbm, invf_hbm, outg_hbm, idxbuf, rows, sem):
        wid = lax.axis_index("s") * 2 + lax.axis_index("c")
        base = wid * SPW
        for h in range(H):
            pltpu.sync_copy(invf_hbm.at[h, pl.ds(base, SPW)], idxbuf)
            pltpu.async_copy(outs_hbm.at[idxbuf], rows, sem).wait()
            pltpu.sync_copy(rows, outg_hbm.at[pl.ds(h * S + base, SPW), :])

    return _sc_scatter, _sc_gather


def _attn_body(qvs_ref, plane_ref, pcol_ref, cnt_ref, out_ref):
    qv = qvs_ref[0]
    q = qv[:, :DH]
    v = qv[:, DH:]
    nrm = jnp.sqrt(jnp.sum(q * q, axis=1, keepdims=True))
    k = q / (nrm + 1e-6)
    qpos = pcol_ref[0]
    kposrow = plane_ref[0]
    cnt = cnt_ref[0, 0]
    for g in range(8):
        lb = (4 * g - 1) % 32
        qg = q[g * 256:(g + 1) * 256]
        kg = jnp.concatenate(
            [k[lb * 64:(lb + 1) * 64], k[g * 256:g * 256 + 256]], axis=0)
        vg = jnp.concatenate(
            [v[lb * 64:(lb + 1) * 64], v[g * 256:g * 256 + 256]], axis=0)
        kp = jnp.concatenate(
            [kposrow[:, lb * 64:(lb + 1) * 64],
             kposrow[:, g * 256:g * 256 + 256]], axis=1)
        sidx = jnp.concatenate(
            [lb * 64 + lax.broadcasted_iota(i32, (1, 64), 1),
             g * 256 + lax.broadcasted_iota(i32, (1, 256), 1)], axis=1)
        s = lax.dot_general(qg, kg, (((1,), (1,)), ((), ())),
                            preferred_element_type=f32) * 0.125
        qp = qpos[g * 256:(g + 1) * 256]
        s = jnp.where(qp >= kp, s, -1e9)
        s = jnp.where(qp == kp, -1e5, s)
        s = jnp.where(sidx < cnt, s, -1e9)
        cc = lax.broadcasted_iota(i32, (256, 320), 1) // 64
        rc = lax.broadcasted_iota(i32, (256, 320), 0) // 64
        s = jnp.where((cc == rc) | (cc == rc + 1), s, -1e30)
        m = jnp.max(s, axis=1, keepdims=True)
        e = jnp.exp(s - m)
        a = e / jnp.sum(e, axis=1, keepdims=True)
        og = jnp.dot(a, vg, preferred_element_type=f32)
        out_ref[0, g * 256:(g + 1) * 256, :] = jnp.concatenate(
            [og, jnp.zeros((256, DH), f32)], axis=1)


def _attn(qvs, perm_lane, perm_col, cnt, interpret=False):
    return pl.pallas_call(
        _attn_body,
        grid=(H,),
        in_specs=[
            pl.BlockSpec((1, S, 2 * DH), lambda h: (h, 0, 0)),
            pl.BlockSpec((1, 1, S), lambda h: (h, 0, 0)),
            pl.BlockSpec((1, S, 1), lambda h: (h, 0, 0)),
            pl.BlockSpec((1, 1), lambda h: (0, 0)),
        ],
        out_specs=pl.BlockSpec((1, S, 2 * DH), lambda h: (h, 0, 0)),
        out_shape=jax.ShapeDtypeStruct((H, S, 2 * DH), f32),
        interpret=interpret,
    )(qvs, perm_lane, perm_col, cnt)


def _oproj_body(x_ref, og_ref, wo_ref, g_ref, b_ref, x1_ref, h2_ref):
    acc = x_ref[...]
    for h in range(H):
        og = og_ref[h * S:(h + 1) * S, :]
        acc = acc + jnp.dot(og[:, :DH], wo_ref[h * DH:(h + 1) * DH, :],
                            preferred_element_type=f32)
    x1_ref[...] = acc
    mu = jnp.mean(acc, axis=1, keepdims=True)
    var = jnp.mean((acc - mu) ** 2, axis=1, keepdims=True)
    h2_ref[...] = (acc - mu) / jnp.sqrt(var + 1e-6) * g_ref[...] + b_ref[...]


def _oproj(x, outg, wo, g, b, interpret=False):
    return pl.pallas_call(
        _oproj_body,
        out_shape=[jax.ShapeDtypeStruct((S, D), f32),
                   jax.ShapeDtypeStruct((S, D), f32)],
        interpret=interpret,
    )(x, outg, wo, g, b)


def _ffn_body(h2_ref, w1_ref, b1_ref, w2_ref, x1_ref, b2_ref, out_ref):
    f = pl.program_id(0)

    @pl.when(f == 0)
    def _():
        out_ref[...] = x1_ref[...] + b2_ref[...]

    t = jnp.dot(h2_ref[...], w1_ref[...], preferred_element_type=f32)
    t = jax.nn.gelu(t + b1_ref[...])
    out_ref[...] += jnp.dot(t, w2_ref[...], preferred_element_type=f32)


def _ffn(h2, w1, b1, w2, x1, b2, interpret=False):
    FB = FF // 4
    return pl.pallas_call(
        _ffn_body,
        grid=(4,),
        in_specs=[
            pl.BlockSpec((S, D), lambda f: (0, 0)),
            pl.BlockSpec((D, FB), lambda f: (0, f)),
            pl.BlockSpec((1, FB), lambda f: (0, f)),
            pl.BlockSpec((FB, D), lambda f: (f, 0)),
            pl.BlockSpec((S, D), lambda f: (0, 0)),
            pl.BlockSpec((1, D), lambda f: (0, 0)),
        ],
        out_specs=pl.BlockSpec((S, D), lambda f: (0, 0)),
        out_shape=jax.ShapeDtypeStruct((S, D), f32),
        interpret=interpret,
    )(h2, w1, b1, w2, x1, b2)


def _final_body(x_ref, lp_ref, g_ref, b_ref, w_ref, lb_ref, out_ref):
    lp = lp_ref[0, 0]
    row = x_ref[pl.ds(lp, 1), :]
    mu = jnp.mean(row, axis=1, keepdims=True)
    var = jnp.mean((row - mu) ** 2, axis=1, keepdims=True)
    rn = (row - mu) / jnp.sqrt(var + 1e-6) * g_ref[...] + b_ref[...]
    logits = jnp.dot(rn, w_ref[...], preferred_element_type=f32) + lb_ref[...]
    m = jnp.max(logits, axis=1, keepdims=True)
    e = jnp.exp(logits - m)
    p = e / jnp.sum(e, axis=1, keepdims=True)
    la = lax.broadcasted_iota(i32, (1, V), 1)
    out_ref[...] = (jnp.sum(jnp.where(la == 59, p, 0.0))
                    - jnp.sum(jnp.where(la == 36, p, 0.0))).reshape(1, 1)


def _final(x, lp, g, b, w, lb, interpret=False):
    return pl.pallas_call(
        _final_body,
        out_shape=jax.ShapeDtypeStruct((1, 1), f32),
        interpret=interpret,
    )(x, lp, g, b, w, lb)


def _permute_sc(qv, invf):
    sc_scatter, _ = _sc_kernels()
    qvs, perm = sc_scatter(qv.reshape(H * S, 2 * DH), invf.reshape(H, S))
    return qvs.reshape(H, S, 2 * DH), perm


def _unpermute_sc(outs, invf):
    _, sc_gather = _sc_kernels()
    return sc_gather(outs.reshape(H * S, 2 * DH), invf.reshape(H, S))


def kernel(input_ids, last_token_pos, tok_emb, ax1, ax2, Wqk, Wv, Wo,
           ln1g, ln1b, ln2g, ln2b, W1, b1, W2, b2, lnfg, lnfb,
           lm_w, lm_b, rot):
    ids_col = input_ids.reshape(S, 1).astype(i32)
    x, cnt = _embed(ids_col, tok_emb, ax1, ax2)
    for l in range(L):
        wqk_h = Wqk[l].reshape(D, H, DH).transpose(1, 0, 2)
        wv_h = Wv[l].reshape(D, H, DH).transpose(1, 0, 2)
        qv, invf = _prep(x, ids_col, ln1g[l].reshape(1, D),
                         ln1b[l].reshape(1, D), wqk_h, wv_h, rot[l])
        qvs, perm = _permute_sc(qv, invf)
        outs = _attn(qvs, perm.reshape(H, 1, S), perm.reshape(H, S, 1), cnt)
        outg = _unpermute_sc(outs, invf)
        x1, h2 = _oproj(x, outg.reshape(H * S, 2 * DH), Wo[l],
                        ln2g[l].reshape(1, D), ln2b[l].reshape(1, D))
        x = _ffn(h2, W1[l], b1[l].reshape(1, FF), W2[l], x1,
                 b2[l].reshape(1, D))
    out = _final(x, last_token_pos.reshape(1, 1).astype(i32),
                 lnfg.reshape(1, D), lnfb.reshape(1, D), lm_w,
                 lm_b.reshape(1, V))
    return out.reshape(B)


# speedup vs baseline: 7.2456x; 1.2256x over previous
import functools

import jax
import jax.numpy as jnp
from jax import lax
from jax.experimental import pallas as pl
from jax.experimental.pallas import tpu as pltpu
from jax.experimental.pallas import tpu_sc as plsc

B, S, D, H, L, FF, V = 1, 2048, 768, 12, 4, 2048, 128
CHUNK = 64
NB = 32
DH = D // H
NW = 32
SPW = S // NW

f32 = jnp.float32
_HI = lax.Precision.HIGHEST
i32 = jnp.int32


def _embed_body(ids_ref, tok_ref, ax1_ref, ax2_ref, x_ref, cnt_ref):
    ids = ids_ref[...]
    lane_v = lax.broadcasted_iota(i32, (S, V), 1)
    oh = (ids == lane_v).astype(f32)
    x = jnp.dot(oh, tok_ref[...], preferred_element_type=f32, precision=_HI)
    su = lax.broadcasted_iota(i32, (S, 32), 0)
    la = lax.broadcasted_iota(i32, (S, 32), 1)
    oh1 = ((su // 64) == la).astype(f32)
    p1 = jnp.dot(oh1, ax1_ref[...], preferred_element_type=f32, precision=_HI)
    su2 = lax.broadcasted_iota(i32, (S, 64), 0)
    la2 = lax.broadcasted_iota(i32, (S, 64), 1)
    oh2 = ((su2 % 64) == la2).astype(f32)
    p2 = jnp.dot(oh2, ax2_ref[...], preferred_element_type=f32, precision=_HI)
    x_ref[...] = x + jnp.concatenate([p1, p2], axis=1)
    cnt_ref[...] = jnp.sum((ids != 0).astype(i32)).reshape(1, 1)


def _embed(ids_col, tok_emb, ax1, ax2, interpret=False):
    return pl.pallas_call(
        _embed_body,
        out_shape=[jax.ShapeDtypeStruct((S, D), f32),
                   jax.ShapeDtypeStruct((1, 1), i32)],
        interpret=interpret,
    )(ids_col, tok_emb, ax1, ax2)


def _prep_body(x_ref, ids_ref, g_ref, b_ref, wqv_ref, rot_ref,
               qv_ref, inv_ref):
    x = x_ref[...]
    mu = jnp.mean(x, axis=1, keepdims=True)
    var = jnp.mean((x - mu) ** 2, axis=1, keepdims=True)
    xn = (x - mu) / jnp.sqrt(var + 1e-6) * g_ref[...] + b_ref[...]
    qv = jnp.dot(xn, wqv_ref[...], preferred_element_type=f32)
    qv_ref[...] = qv
    la32 = lax.broadcasted_iota(i32, (S, NB), 1)
    la64 = lax.broadcasted_iota(i32, (S, 64), 1)
    notpad = ids_ref[...] != 0
    ohs = []
    for h in range(H):
        q = qv[:, h * 2 * DH:h * 2 * DH + DH]
        proj = jnp.dot(q, rot_ref[h], preferred_element_type=f32)
        c = jnp.concatenate([proj, -proj], axis=1)
        m = jnp.max(c, axis=1, keepdims=True)
        bidx = jnp.min(jnp.where(c == m, la32, NB * 2), axis=1, keepdims=True)
        bucket = jnp.where(notpad, bidx, NB)
        ohs.append((bucket == la64).astype(f32))
    oh = jnp.concatenate(ohs, axis=1)
    counts = jnp.sum(oh, axis=0, keepdims=True)
    br = lax.broadcasted_iota(i32, (D, D), 0)
    bc = lax.broadcasted_iota(i32, (D, D), 1)
    strict_bd = ((br // 64 == bc // 64) & (br % 64 < bc % 64)).astype(f32)
    start = jnp.dot(counts, strict_bd, preferred_element_type=f32,
                    precision=lax.Precision.HIGHEST)
    gr = lax.broadcasted_iota(i32, (D, 16), 0)
    gc = lax.broadcasted_iota(i32, (D, 16), 1)
    gmat = (gr // 64 == gc).astype(f32)
    offs = (lax.broadcasted_iota(i32, (1, 16), 1) * S).astype(f32)
    tr = lax.broadcasted_iota(i32, (128, 128), 0)
    tc = lax.broadcasted_iota(i32, (128, 128), 1)
    tri = (tr >= tc).astype(f32)
    carry = jnp.zeros((1, D), f32)
    invs = []
    for i in range(S // 128):
        blk = oh[i * 128:(i + 1) * 128]
        cum = jnp.dot(tri, blk, preferred_element_type=f32) + carry
        t = (cum + start) * blk
        invs.append(jnp.dot(t, gmat, preferred_element_type=f32,
                            precision=lax.Precision.HIGHEST)
                    - 1.0 + offs)
        carry = carry + jnp.sum(blk, axis=0, keepdims=True)
    inv2 = jnp.concatenate(invs, axis=0)
    inv_ref[...] = jnp.transpose(inv2).astype(i32)


def _prep(x, ids_col, g, b, wqv, rot_l, interpret=False):
    return pl.pallas_call(
        _prep_body,
        out_shape=[jax.ShapeDtypeStruct((S, 2 * DH * H), f32),
                   jax.ShapeDtypeStruct((16, S), i32)],
        interpret=interpret,
    )(x, ids_col, g, b, wqv, rot_l)


@functools.lru_cache(maxsize=1)
def _sc_kernels():
    mesh = plsc.VectorSubcoreMesh(core_axis_name="c", subcore_axis_name="s")

    scr = ([pltpu.VMEM((SPW,), i32) for _ in range(H)]
           + [pltpu.VMEM((SPW, 2 * DH), f32) for _ in range(H)]
           + [pltpu.VMEM((S,), i32), pltpu.VMEM((S,), i32),
              pltpu.SemaphoreType.DMA, pltpu.SemaphoreType.DMA])

    @functools.partial(
        pl.kernel,
        out_type=[jax.ShapeDtypeStruct((H * S, 2 * DH), f32),
                  jax.ShapeDtypeStruct((H, S), i32)],
        mesh=mesh,
        scratch_types=scr,
        compiler_params=pltpu.CompilerParams(needs_layout_passes=False),
    )
    def _sc_scatter(qv_hbm, invf_hbm, qvs_hbm, perm_hbm, *scratch):
        idxb = scratch[:H]
        rowb = scratch[H:2 * H]
        invfull, permbuf, sem_a, sem_b = scratch[2 * H:]
        wid = lax.axis_index("s") * 2 + lax.axis_index("c")
        base = wid * SPW
        cps = [pltpu.async_copy(invf_hbm.at[h, pl.ds(base, SPW)],
                                idxb[h], sem_a) for h in range(H)]
        cps += [pltpu.async_copy(
            qv_hbm.at[pl.ds(base, SPW), pl.ds(h * 2 * DH, 2 * DH)],
            rowb[h], sem_b) for h in range(H)]
        for c in cps:
            c.wait()
        cps = [pltpu.async_copy(rowb[h], qvs_hbm.at[idxb[h]], sem_a)
               for h in range(H)]

        @pl.when(wid < H)
        def _():
            pltpu.sync_copy(invf_hbm.at[wid], invfull)
            for i in range(S // 16):
                idx16 = invfull[pl.ds(i * 16, 16)] - wid * S
                vals = lax.iota(i32, 16) + i * 16
                plsc.store_scatter(permbuf, [idx16], vals)
            pltpu.sync_copy(permbuf, perm_hbm.at[wid])

        for c in cps:
            c.wait()

    @functools.partial(
        pl.kernel,
        out_type=jax.ShapeDtypeStruct((H * S, 2 * DH), f32),
        mesh=mesh,
        scratch_types=scr[:2 * H] + scr[-2:],
        compiler_params=pltpu.CompilerParams(needs_layout_passes=False),
    )
    def _sc_gather(outs_hbm, invf_hbm, outg_hbm, *scratch):
        idxb = scratch[:H]
        rowb = scratch[H:2 * H]
        sem_a, sem_b = scratch[2 * H:]
        wid = lax.axis_index("s") * 2 + lax.axis_index("c")
        base = wid * SPW
        cps = [pltpu.async_copy(invf_hbm.at[h, pl.ds(base, SPW)],
                                idxb[h], sem_a) for h in range(H)]
        for c in cps:
            c.wait()
        cps = [pltpu.async_copy(outs_hbm.at[idxb[h]], rowb[h], sem_b)
               for h in range(H)]
        for c in cps:
            c.wait()
        cps = [pltpu.async_copy(rowb[h],
                                outg_hbm.at[pl.ds(h * S + base, SPW), :],
                                sem_a) for h in range(H)]
        for c in cps:
            c.wait()

    return _sc_scatter, _sc_gather


def _attn_body(qvs_ref, plane_ref, pcol_ref, cnt_ref, out_ref):
    qv = qvs_ref[0]
    q = qv[:, :DH]
    v = qv[:, DH:]
    nrm = jnp.sqrt(jnp.sum(q * q, axis=1, keepdims=True))
    k = q / (nrm + 1e-6)
    qpos = pcol_ref[0]
    kposrow = plane_ref[0]
    cnt = cnt_ref[0, 0]
    for g in range(8):
        lb = (4 * g - 1) % 32
        qg = q[g * 256:(g + 1) * 256]
        kg = jnp.concatenate(
            [k[lb * 64:(lb + 1) * 64], k[g * 256:g * 256 + 256]], axis=0)
        vg = jnp.concatenate(
            [v[lb * 64:(lb + 1) * 64], v[g * 256:g * 256 + 256]], axis=0)
        kp = jnp.concatenate(
            [kposrow[:, lb * 64:(lb + 1) * 64],
             kposrow[:, g * 256:g * 256 + 256]], axis=1)
        sidx = jnp.concatenate(
            [lb * 64 + lax.broadcasted_iota(i32, (1, 64), 1),
             g * 256 + lax.broadcasted_iota(i32, (1, 256), 1)], axis=1)
        s = lax.dot_general(qg, kg, (((1,), (1,)), ((), ())),
                            preferred_element_type=f32) * 0.125
        qp = qpos[g * 256:(g + 1) * 256]
        s = jnp.where(qp >= kp, s, -1e9)
        s = jnp.where(qp == kp, -1e5, s)
        s = jnp.where(sidx < cnt, s, -1e9)
        cc = lax.broadcasted_iota(i32, (256, 320), 1) // 64
        rc = lax.broadcasted_iota(i32, (256, 320), 0) // 64
        s = jnp.where((cc == rc) | (cc == rc + 1), s, -1e30)
        m = jnp.max(s, axis=1, keepdims=True)
        e = jnp.exp(s - m)
        a = e / jnp.sum(e, axis=1, keepdims=True)
        og = jnp.dot(a, vg, preferred_element_type=f32)
        out_ref[0, g * 256:(g + 1) * 256, :] = jnp.concatenate(
            [og, jnp.zeros((256, DH), f32)], axis=1)


def _attn(qvs, perm_lane, perm_col, cnt, interpret=False):
    return pl.pallas_call(
        _attn_body,
        grid=(H,),
        in_specs=[
            pl.BlockSpec((1, S, 2 * DH), lambda h: (h, 0, 0)),
            pl.BlockSpec((1, 1, S), lambda h: (h, 0, 0)),
            pl.BlockSpec((1, S, 1), lambda h: (h, 0, 0)),
            pl.BlockSpec((1, 1), lambda h: (0, 0)),
        ],
        out_specs=pl.BlockSpec((1, S, 2 * DH), lambda h: (h, 0, 0)),
        out_shape=jax.ShapeDtypeStruct((H, S, 2 * DH), f32),
        interpret=interpret,
    )(qvs, perm_lane, perm_col, cnt)


def _oproj_body(x_ref, og_ref, wo_ref, g_ref, b_ref, x1_ref, h2_ref):
    h = pl.program_id(0)

    @pl.when(h == 0)
    def _():
        x1_ref[...] = x_ref[...]

    og = og_ref[0]
    x1_ref[...] += jnp.dot(og[:, :DH], wo_ref[0],
                           preferred_element_type=f32)

    @pl.when(h == H - 1)
    def _():
        acc = x1_ref[...]
        mu = jnp.mean(acc, axis=1, keepdims=True)
        var = jnp.mean((acc - mu) ** 2, axis=1, keepdims=True)
        h2_ref[...] = ((acc - mu) / jnp.sqrt(var + 1e-6) * g_ref[...]
                       + b_ref[...])


def _oproj(x, outg, wo, g, b, interpret=False):
    return pl.pallas_call(
        _oproj_body,
        grid=(H,),
        in_specs=[
            pl.BlockSpec((S, D), lambda h: (0, 0)),
            pl.BlockSpec((1, S, 2 * DH), lambda h: (h, 0, 0)),
            pl.BlockSpec((1, DH, D), lambda h: (h, 0, 0)),
            pl.BlockSpec((1, D), lambda h: (0, 0)),
            pl.BlockSpec((1, D), lambda h: (0, 0)),
        ],
        out_specs=[pl.BlockSpec((S, D), lambda h: (0, 0)),
                   pl.BlockSpec((S, D), lambda h: (0, 0))],
        out_shape=[jax.ShapeDtypeStruct((S, D), f32),
                   jax.ShapeDtypeStruct((S, D), f32)],
        interpret=interpret,
    )(x, outg.reshape(H, S, 2 * DH), wo.reshape(H, DH, D), g, b)


def _ffn_body(h2_ref, w1_ref, b1_ref, w2_ref, x1_ref, b2_ref, out_ref):
    f = pl.program_id(0)

    @pl.when(f == 0)
    def _():
        out_ref[...] = x1_ref[...] + b2_ref[...]

    t = jnp.dot(h2_ref[...], w1_ref[...], preferred_element_type=f32)
    t = jax.nn.gelu(t + b1_ref[...])
    out_ref[...] += jnp.dot(t, w2_ref[...], preferred_element_type=f32)


def _ffn(h2, w1, b1, w2, x1, b2, interpret=False):
    FB = FF // 4
    return pl.pallas_call(
        _ffn_body,
        grid=(4,),
        in_specs=[
            pl.BlockSpec((S, D), lambda f: (0, 0)),
            pl.BlockSpec((D, FB), lambda f: (0, f)),
            pl.BlockSpec((1, FB), lambda f: (0, f)),
            pl.BlockSpec((FB, D), lambda f: (f, 0)),
            pl.BlockSpec((S, D), lambda f: (0, 0)),
            pl.BlockSpec((1, D), lambda f: (0, 0)),
        ],
        out_specs=pl.BlockSpec((S, D), lambda f: (0, 0)),
        out_shape=jax.ShapeDtypeStruct((S, D), f32),
        interpret=interpret,
    )(h2, w1, b1, w2, x1, b2)


def _final_body(x_ref, lp_ref, g_ref, b_ref, w_ref, lb_ref, out_ref):
    lp = lp_ref[0, 0]
    row = x_ref[pl.ds(lp, 1), :]
    mu = jnp.mean(row, axis=1, keepdims=True)
    var = jnp.mean((row - mu) ** 2, axis=1, keepdims=True)
    rn = (row - mu) / jnp.sqrt(var + 1e-6) * g_ref[...] + b_ref[...]
    logits = jnp.dot(rn, w_ref[...], preferred_element_type=f32) + lb_ref[...]
    m = jnp.max(logits, axis=1, keepdims=True)
    e = jnp.exp(logits - m)
    p = e / jnp.sum(e, axis=1, keepdims=True)
    la = lax.broadcasted_iota(i32, (1, V), 1)
    out_ref[...] = (jnp.sum(jnp.where(la == 59, p, 0.0))
                    - jnp.sum(jnp.where(la == 36, p, 0.0))).reshape(1, 1)


def _final(x, lp, g, b, w, lb, interpret=False):
    return pl.pallas_call(
        _final_body,
        out_shape=jax.ShapeDtypeStruct((1, 1), f32),
        interpret=interpret,
    )(x, lp, g, b, w, lb)


def _permute_sc(qv, invf):
    sc_scatter, _ = _sc_kernels()
    qvs, perm = sc_scatter(qv, invf)
    return qvs.reshape(H, S, 2 * DH), perm


def _unpermute_sc(outs, invf):
    _, sc_gather = _sc_kernels()
    return sc_gather(outs.reshape(H * S, 2 * DH), invf)


def kernel(input_ids, last_token_pos, tok_emb, ax1, ax2, Wqk, Wv, Wo,
           ln1g, ln1b, ln2g, ln2b, W1, b1, W2, b2, lnfg, lnfb,
           lm_w, lm_b, rot):
    ids_col = input_ids.reshape(S, 1).astype(i32)
    x, cnt = _embed(ids_col, tok_emb, ax1, ax2)
    for l in range(L):
        wqv = jnp.concatenate(
            [Wqk[l].reshape(D, H, DH), Wv[l].reshape(D, H, DH)],
            axis=2).reshape(D, 2 * DH * H)
        qv, invf = _prep(x, ids_col, ln1g[l].reshape(1, D),
                         ln1b[l].reshape(1, D), wqv, rot[l])
        qvs, perm = _permute_sc(qv, invf)
        outs = _attn(qvs, perm.reshape(H, 1, S), perm.reshape(H, S, 1), cnt)
        outg = _unpermute_sc(outs, invf)
        x1, h2 = _oproj(x, outg.reshape(H * S, 2 * DH), Wo[l],
                        ln2g[l].reshape(1, D), ln2b[l].reshape(1, D))
        x = _ffn(h2, W1[l], b1[l].reshape(1, FF), W2[l], x1,
                 b2[l].reshape(1, D))
    out = _final(x, last_token_pos.reshape(1, 1).astype(i32),
                 lnfg.reshape(1, D), lnfb.reshape(1, D), lm_w,
                 lm_b.reshape(1, V))
    return out.reshape(B)
